# baseline (speedup 1.0000x reference)
import functools

import jax
import jax.numpy as jnp
from jax import lax
from jax.experimental import pallas as pl
from jax.experimental.pallas import tpu as pltpu
from jax.experimental.pallas import tpu_sc as plsc

_BT = 128
_NC = 2
_NS = 16
_NSPLIT = 8


def _router_body(x_ref, wg_ref, pos_ref, texp_ref, tact_ref, rid_ref,
                 rexp_ref, nrun_ref, *, BT, NT):
    T = x_ref.shape[0]
    E = wg_ref.shape[1]
    x = x_ref[...]
    wg = wg_ref[...]
    logits = jnp.dot(x, wg, preferred_element_type=jnp.float32)
    eidx = jnp.argmax(logits, axis=1, keepdims=True)
    eids = lax.broadcasted_iota(jnp.int32, (T, E), 1)
    onehot = (eids == eidx).astype(jnp.float32)
    rowi = lax.broadcasted_iota(jnp.int32, (T, T), 0)
    coli = lax.broadcasted_iota(jnp.int32, (T, T), 1)
    ltri = (coli < rowi).astype(jnp.float32)
    prefix = jnp.dot(ltri, onehot, preferred_element_type=jnp.float32)
    rank = jnp.sum(prefix * onehot, axis=1, keepdims=True)
    counts = jnp.sum(onehot, axis=0, keepdims=True).astype(jnp.int32)
    ntile = (counts + BT - 1) // BT
    ar = lax.broadcasted_iota(jnp.int32, (E, E), 0)
    bc = lax.broadcasted_iota(jnp.int32, (E, E), 1)
    umaskf = (ar < bc).astype(jnp.float32)
    tstart = jnp.dot(ntile.astype(jnp.float32), umaskf,
                     preferred_element_type=jnp.float32)
    startrow = tstart * BT
    posf = jnp.sum(onehot * startrow, axis=1, keepdims=True) + rank
    pos_ref[...] = posf.astype(jnp.int32)
    total = jnp.sum(ntile, axis=1, keepdims=True)
    tstart_i = tstart.astype(jnp.int32)
    ti = lax.broadcasted_iota(jnp.int32, (NT, E), 0)
    te = lax.broadcasted_iota(jnp.int32, (NT, E), 1)
    inr = (ti >= tstart_i) & (ti < tstart_i + ntile)
    texp = jnp.sum(jnp.where(inr, te, 0), axis=1, keepdims=True)
    tid = lax.broadcasted_iota(jnp.int32, (NT, 1), 0)
    active = tid < total
    e_last = jnp.sum(jnp.where(tid == total - 1, texp, 0), axis=0,
                     keepdims=True)
    texp_ref[...] = jnp.where(active, texp, e_last)
    tact_ref[...] = active.astype(jnp.int32)
    used = (counts > 0).astype(jnp.float32)
    used_rank = jnp.dot(used, umaskf,
                        preferred_element_type=jnp.float32).astype(jnp.int32)
    nruns = jnp.sum((counts > 0).astype(jnp.int32), axis=1, keepdims=True)
    ridc = jnp.sum(jnp.where(inr, used_rank, 0), axis=1, keepdims=True)
    rid_ref[...] = jnp.where(active, ridc, nruns - 1)
    rr = lax.broadcasted_iota(jnp.int32, (NT, E), 0)
    sel = (rr == used_rank) & (counts > 0)
    rexp_ref[...] = jnp.sum(jnp.where(sel, te, 0), axis=1, keepdims=True)
    nrun_ref[...] = jnp.broadcast_to(nruns, (1, 1))


def _ffn_body(texp_ref, tact_ref, rid_ref, rexp_ref, nrun_ref,
              xs_ref, w1_hbm, b1_ref, w2_hbm, b2_ref, out_ref,
              w1buf, w2buf, sem1, sem2, *, NSPLIT):
    i = pl.program_id(0)
    d = w1_hbm.shape[1]
    H = w1_hbm.shape[2]
    dS = d // NSPLIT
    hS = H // NSPLIT
    nruns = nrun_ref[0]
    r = rid_ref[i]
    slot = lax.rem(r, 2)

    def issue(e, sl):
        for s in range(NSPLIT):
            pltpu.make_async_copy(
                w1_hbm.at[e, pl.ds(s * dS, dS), :],
                w1buf.at[sl, pl.ds(s * dS, dS), :],
                sem1.at[sl, s]).start()
            pltpu.make_async_copy(
                w2_hbm.at[e, pl.ds(s * hS, hS), :],
                w2buf.at[sl, pl.ds(s * hS, hS), :],
                sem2.at[sl, s]).start()

    def drain(e, sl):
        for s in range(NSPLIT):
            pltpu.make_async_copy(
                w1_hbm.at[e, pl.ds(s * dS, dS), :],
                w1buf.at[sl, pl.ds(s * dS, dS), :],
                sem1.at[sl, s]).wait()
            pltpu.make_async_copy(
                w2_hbm.at[e, pl.ds(s * hS, hS), :],
                w2buf.at[sl, pl.ds(s * hS, hS), :],
                sem2.at[sl, s]).wait()

    j = jnp.maximum(i - 1, 0)
    first = jnp.logical_or(i == 0, rid_ref[j] != r)

    @pl.when(jnp.logical_and(first, r == 0))
    def _():
        issue(rexp_ref[0], 0)

    @pl.when(jnp.logical_and(first, r + 1 < nruns))
    def _():
        issue(rexp_ref[r + 1], lax.rem(r + 1, 2))

    @pl.when(first)
    def _():
        drain(rexp_ref[r], slot)

    @pl.when(tact_ref[i] == 1)
    def _():
        xt = xs_ref[...]
        h = jnp.dot(xt, w1buf[slot], preferred_element_type=jnp.float32)
        h = jnp.maximum(h + b1_ref[0], 0.0)
        y = jnp.dot(h, w2buf[slot], preferred_element_type=jnp.float32)
        out_ref[...] = y + b2_ref[0]


def _sc_dispatch(xr, pos, rows_out):
    T, d = xr.shape
    NW = _NC * _NS
    chunk = T // NW
    mesh = plsc.VectorSubcoreMesh(core_axis_name="c", subcore_axis_name="s")

    @functools.partial(
        pl.kernel, mesh=mesh,
        out_type=jax.ShapeDtypeStruct((rows_out, d), jnp.float32),
        scratch_types=[
            pltpu.VMEM((chunk,), jnp.int32),
            pltpu.VMEM((chunk, d), jnp.float32),
            pltpu.SemaphoreType.DMA,
        ],
    )
    def run(x_hbm, pos_hbm, out_hbm, idx_v, rows_v, sem):
        wid = lax.axis_index("s") * _NC + lax.axis_index("c")
        base = wid * chunk
        pltpu.sync_copy(pos_hbm.at[pl.ds(base, chunk)], idx_v)
        pltpu.sync_copy(x_hbm.at[pl.ds(base, chunk)], rows_v)
        pltpu.async_copy(rows_v, out_hbm.at[idx_v], sem).wait()

    return run(xr, pos)


def _sc_combine(ys, pos, T):
    d = ys.shape[1]
    NW = _NC * _NS
    chunk = T // NW
    mesh = plsc.VectorSubcoreMesh(core_axis_name="c", subcore_axis_name="s")

    @functools.partial(
        pl.kernel, mesh=mesh,
        out_type=jax.ShapeDtypeStruct((T, d), jnp.float32),
        scratch_types=[
            pltpu.VMEM((chunk,), jnp.int32),
            pltpu.VMEM((chunk, d), jnp.float32),
            pltpu.SemaphoreType.DMA,
        ],
    )
    def run(ys_hbm, pos_hbm, out_hbm, idx_v, rows_v, sem):
        wid = lax.axis_index("s") * _NC + lax.axis_index("c")
        base = wid * chunk
        pltpu.sync_copy(pos_hbm.at[pl.ds(base, chunk)], idx_v)
        pltpu.async_copy(ys_hbm.at[idx_v], rows_v, sem).wait()
        pltpu.sync_copy(rows_v, out_hbm.at[pl.ds(base, chunk)])

    return run(ys, pos)


def kernel(x, Wg, W1, b1, W2, b2):
    B, S, d = x.shape
    E = Wg.shape[1]
    H = W1.shape[2]
    T = B * S
    BT = _BT
    NT = T // BT + E
    xr = x.reshape(T, d)

    pos2, texp2, tact2, rid2, rexp2, nrun2 = pl.pallas_call(
        functools.partial(_router_body, BT=BT, NT=NT),
        out_shape=[
            jax.ShapeDtypeStruct((T, 1), jnp.int32),
            jax.ShapeDtypeStruct((NT, 1), jnp.int32),
            jax.ShapeDtypeStruct((NT, 1), jnp.int32),
            jax.ShapeDtypeStruct((NT, 1), jnp.int32),
            jax.ShapeDtypeStruct((NT, 1), jnp.int32),
            jax.ShapeDtypeStruct((1, 1), jnp.int32),
        ],
    )(xr, Wg)
    pos = pos2.reshape(T)

    xs = _sc_dispatch(xr, pos, NT * BT)

    grid_spec = pltpu.PrefetchScalarGridSpec(
        num_scalar_prefetch=5,
        grid=(NT,),
        in_specs=[
            pl.BlockSpec((BT, d), lambda i, *_: (i, 0)),
            pl.BlockSpec(memory_space=pl.ANY),
            pl.BlockSpec((1, 1, H), lambda i, te, *_: (te[i], 0, 0)),
            pl.BlockSpec(memory_space=pl.ANY),
            pl.BlockSpec((1, 1, d), lambda i, te, *_: (te[i], 0, 0)),
        ],
        out_specs=pl.BlockSpec((BT, d), lambda i, *_: (i, 0)),
        scratch_shapes=[
            pltpu.VMEM((2, d, H), jnp.float32),
            pltpu.VMEM((2, H, d), jnp.float32),
            pltpu.SemaphoreType.DMA((2, _NSPLIT)),
            pltpu.SemaphoreType.DMA((2, _NSPLIT)),
        ],
    )
    ys = pl.pallas_call(
        functools.partial(_ffn_body, NSPLIT=_NSPLIT),
        grid_spec=grid_spec,
        out_shape=jax.ShapeDtypeStruct((NT * BT, d), jnp.float32),
    )(texp2.reshape(NT), tact2.reshape(NT), rid2.reshape(NT),
      rexp2.reshape(NT), nrun2.reshape(1), xs, W1,
      b1.reshape(E, 1, H), W2, b2.reshape(E, 1, d))

    out = _sc_combine(ys, pos, T)
    return out.reshape(B, S, d)


# speedup vs baseline: 1.0545x; 1.0545x over previous
import functools

import jax
import jax.numpy as jnp
from jax import lax
from jax.experimental import pallas as pl
from jax.experimental.pallas import tpu as pltpu
from jax.experimental.pallas import tpu_sc as plsc

_BT = 128
_NC = 2
_NS = 16
_NSPLIT = 4


def _router_body(x_ref, wg_ref, pos_ref, texp_ref, tact_ref, rid_ref,
                 rexp_ref, nrun_ref, ctile_ref, *, BT, NT):
    T = x_ref.shape[0]
    E = wg_ref.shape[1]
    x = x_ref[...]
    wg = wg_ref[...]
    logits = jnp.dot(x, wg, preferred_element_type=jnp.float32)
    eidx = jnp.argmax(logits, axis=1, keepdims=True)
    eids = lax.broadcasted_iota(jnp.int32, (T, E), 1)
    onehot = (eids == eidx).astype(jnp.float32)
    rowi = lax.broadcasted_iota(jnp.int32, (T, T), 0)
    coli = lax.broadcasted_iota(jnp.int32, (T, T), 1)
    ltri = (coli < rowi).astype(jnp.float32)
    prefix = jnp.dot(ltri, onehot, preferred_element_type=jnp.float32)
    rank = jnp.sum(prefix * onehot, axis=1, keepdims=True)
    counts = jnp.sum(onehot, axis=0, keepdims=True).astype(jnp.int32)
    ntile = (counts + BT - 1) // BT
    ar = lax.broadcasted_iota(jnp.int32, (E, E), 0)
    bc = lax.broadcasted_iota(jnp.int32, (E, E), 1)
    umaskf = (ar < bc).astype(jnp.float32)
    tstart = jnp.dot(ntile.astype(jnp.float32), umaskf,
                     preferred_element_type=jnp.float32)
    startrow = tstart * BT
    posf = jnp.sum(onehot * startrow, axis=1, keepdims=True) + rank
    pos_ref[...] = posf.astype(jnp.int32)
    total = jnp.sum(ntile, axis=1, keepdims=True)
    tstart_i = tstart.astype(jnp.int32)
    ti = lax.broadcasted_iota(jnp.int32, (NT, E), 0)
    te = lax.broadcasted_iota(jnp.int32, (NT, E), 1)
    inr = (ti >= tstart_i) & (ti < tstart_i + ntile)
    texp = jnp.sum(jnp.where(inr, te, 0), axis=1, keepdims=True)
    tid = lax.broadcasted_iota(jnp.int32, (NT, 1), 0)
    active = tid < total
    e_last = jnp.sum(jnp.where(tid == total - 1, texp, 0), axis=0,
                     keepdims=True)
    texp_ref[...] = jnp.where(active, texp, e_last)
    tact_ref[...] = active.astype(jnp.int32)
    used = (counts > 0).astype(jnp.float32)
    used_rank = jnp.dot(used, umaskf,
                        preferred_element_type=jnp.float32).astype(jnp.int32)
    nruns = jnp.sum((counts > 0).astype(jnp.int32), axis=1, keepdims=True)
    ridc = jnp.sum(jnp.where(inr, used_rank, 0), axis=1, keepdims=True)
    rid_ref[...] = jnp.where(active, ridc, nruns - 1)
    rr = lax.broadcasted_iota(jnp.int32, (NT, E), 0)
    sel = (rr == used_rank) & (counts > 0)
    rexp_ref[...] = jnp.sum(jnp.where(sel, te, 0), axis=1, keepdims=True)
    nrun_ref[...] = jnp.broadcast_to(nruns, (1, 1))
    ctile_ref[...] = jnp.minimum(tid, total - 1)


def _ffn_body(texp_ref, tact_ref, rid_ref, rexp_ref, nrun_ref, ctile_ref,
              xs_ref, w1_hbm, b1_ref, w2_hbm, b2_ref, out_ref,
              w1buf, w2buf, sem1, sem2, *, NSPLIT):
    i = pl.program_id(0)
    d = w1_hbm.shape[1]
    H = w1_hbm.shape[2]
    dS = d // NSPLIT
    hS = H // NSPLIT
    nruns = nrun_ref[0]
    r = rid_ref[i]
    slot = lax.rem(r, 2)

    def issue(e, sl):
        for s in range(NSPLIT):
            pltpu.make_async_copy(
                w1_hbm.at[e, pl.ds(s * dS, dS), :],
                w1buf.at[sl, pl.ds(s * dS, dS), :],
                sem1.at[sl, s]).start()
            pltpu.make_async_copy(
                w2_hbm.at[e, pl.ds(s * hS, hS), :],
                w2buf.at[sl, pl.ds(s * hS, hS), :],
                sem2.at[sl, s]).start()

    def drain(e, sl):
        for s in range(NSPLIT):
            pltpu.make_async_copy(
                w1_hbm.at[e, pl.ds(s * dS, dS), :],
                w1buf.at[sl, pl.ds(s * dS, dS), :],
                sem1.at[sl, s]).wait()
            pltpu.make_async_copy(
                w2_hbm.at[e, pl.ds(s * hS, hS), :],
                w2buf.at[sl, pl.ds(s * hS, hS), :],
                sem2.at[sl, s]).wait()

    j = jnp.maximum(i - 1, 0)
    first = jnp.logical_or(i == 0, rid_ref[j] != r)

    @pl.when(jnp.logical_and(first, r == 0))
    def _():
        issue(rexp_ref[0], 0)

    @pl.when(jnp.logical_and(first, r + 1 < nruns))
    def _():
        issue(rexp_ref[r + 1], lax.rem(r + 1, 2))

    @pl.when(first)
    def _():
        drain(rexp_ref[r], slot)

    @pl.when(tact_ref[i] == 1)
    def _():
        xt = xs_ref[...]
        h = jnp.dot(xt, w1buf[slot], preferred_element_type=jnp.float32)
        h = jnp.maximum(h + b1_ref[0], 0.0)
        y = jnp.dot(h, w2buf[slot], preferred_element_type=jnp.float32)
        out_ref[...] = y + b2_ref[0]


def _sc_dispatch(xr, pos, rows_out):
    T, d = xr.shape
    NW = _NC * _NS
    chunk = T // NW
    mesh = plsc.VectorSubcoreMesh(core_axis_name="c", subcore_axis_name="s")

    @functools.partial(
        pl.kernel, mesh=mesh,
        out_type=jax.ShapeDtypeStruct((rows_out, d), jnp.float32),
        scratch_types=[
            pltpu.VMEM((chunk,), jnp.int32),
            pltpu.VMEM((chunk, d), jnp.float32),
            pltpu.SemaphoreType.DMA((2,)),
        ],
    )
    def run(x_hbm, pos_hbm, out_hbm, idx_v, rows_v, sem):
        wid = lax.axis_index("s") * _NC + lax.axis_index("c")
        base = wid * chunk
        c1 = pltpu.async_copy(pos_hbm.at[pl.ds(base, chunk)], idx_v,
                              sem.at[0])
        c2 = pltpu.async_copy(x_hbm.at[pl.ds(base, chunk)], rows_v,
                              sem.at[1])
        c1.wait()
        c2.wait()
        pltpu.async_copy(rows_v, out_hbm.at[idx_v], sem.at[0]).wait()

    return run(xr, pos)


def _sc_combine(ys, pos, T):
    d = ys.shape[1]
    NW = _NC * _NS
    chunk = T // NW
    mesh = plsc.VectorSubcoreMesh(core_axis_name="c", subcore_axis_name="s")

    @functools.partial(
        pl.kernel, mesh=mesh,
        out_type=jax.ShapeDtypeStruct((T, d), jnp.float32),
        scratch_types=[
            pltpu.VMEM((chunk,), jnp.int32),
            pltpu.VMEM((chunk, d), jnp.float32),
            pltpu.SemaphoreType.DMA,
        ],
    )
    def run(ys_hbm, pos_hbm, out_hbm, idx_v, rows_v, sem):
        wid = lax.axis_index("s") * _NC + lax.axis_index("c")
        base = wid * chunk
        pltpu.sync_copy(pos_hbm.at[pl.ds(base, chunk)], idx_v)
        pltpu.async_copy(ys_hbm.at[idx_v], rows_v, sem).wait()
        pltpu.sync_copy(rows_v, out_hbm.at[pl.ds(base, chunk)])

    return run(ys, pos)


def kernel(x, Wg, W1, b1, W2, b2):
    B, S, d = x.shape
    E = Wg.shape[1]
    H = W1.shape[2]
    T = B * S
    BT = _BT
    NT = T // BT + E
    xr = x.reshape(T, d)

    pos2, texp2, tact2, rid2, rexp2, nrun2, ct2 = pl.pallas_call(
        functools.partial(_router_body, BT=BT, NT=NT),
        out_shape=[
            jax.ShapeDtypeStruct((T, 1), jnp.int32),
            jax.ShapeDtypeStruct((NT, 1), jnp.int32),
            jax.ShapeDtypeStruct((NT, 1), jnp.int32),
            jax.ShapeDtypeStruct((NT, 1), jnp.int32),
            jax.ShapeDtypeStruct((NT, 1), jnp.int32),
            jax.ShapeDtypeStruct((1, 1), jnp.int32),
            jax.ShapeDtypeStruct((NT, 1), jnp.int32),
        ],
    )(xr, Wg)
    pos = pos2.reshape(T)

    xs = _sc_dispatch(xr, pos, NT * BT)

    grid_spec = pltpu.PrefetchScalarGridSpec(
        num_scalar_prefetch=6,
        grid=(NT,),
        in_specs=[
            pl.BlockSpec((BT, d),
                         lambda i, te, ta, ri, re, nr, ct: (ct[i], 0)),
            pl.BlockSpec(memory_space=pl.ANY),
            pl.BlockSpec((1, 1, H),
                         lambda i, te, ta, ri, re, nr, ct: (te[i], 0, 0)),
            pl.BlockSpec(memory_space=pl.ANY),
            pl.BlockSpec((1, 1, d),
                         lambda i, te, ta, ri, re, nr, ct: (te[i], 0, 0)),
        ],
        out_specs=pl.BlockSpec((BT, d),
                               lambda i, te, ta, ri, re, nr, ct: (ct[i], 0)),
        scratch_shapes=[
            pltpu.VMEM((2, d, H), jnp.float32),
            pltpu.VMEM((2, H, d), jnp.float32),
            pltpu.SemaphoreType.DMA((2, _NSPLIT)),
            pltpu.SemaphoreType.DMA((2, _NSPLIT)),
        ],
    )
    ys = pl.pallas_call(
        functools.partial(_ffn_body, NSPLIT=_NSPLIT),
        grid_spec=grid_spec,
        out_shape=jax.ShapeDtypeStruct((NT * BT, d), jnp.float32),
    )(texp2.reshape(NT), tact2.reshape(NT), rid2.reshape(NT),
      rexp2.reshape(NT), nrun2.reshape(1), ct2.reshape(NT), xs, W1,
      b1.reshape(E, 1, H), W2, b2.reshape(E, 1, d))

    out = _sc_combine(ys, pos, T)
    return out.reshape(B, S, d)
